# baseline (speedup 1.0000x reference)
import functools

import jax
import jax.numpy as jnp
from jax import lax
from jax.experimental import pallas as pl
from jax.experimental.pallas import tpu as pltpu
from jax.experimental.pallas import tpu_sc as plsc

_N = 10000
_E = 640000
_D = 128
_HOPS = 3
_NC = 2
_NS = 16
_NW = _NC * _NS
_C = 128
_SB = 8
_NSUP = 60
_CPW = _SB * _NSUP
_NCHUNK = _NW * _CPW
_EPAD = _NCHUNK * _C
_TOTAL_E = _HOPS * _E
_MB = 2000


def _matmul_body(x_ref, w_ref, out_ref):
    out_ref[0] = jnp.dot(x_ref[...], w_ref[0],
                         preferred_element_type=jnp.float32)


def _feats(x, W):
    return pl.pallas_call(
        _matmul_body,
        grid=(_HOPS, _N // _MB),
        in_specs=[
            pl.BlockSpec((_MB, _D), lambda h, i: (i, 0)),
            pl.BlockSpec((1, _D, _D), lambda h, i: (h, 0, 0)),
        ],
        out_specs=pl.BlockSpec((1, _MB, _D), lambda h, i: (h, i, 0)),
        out_shape=jax.ShapeDtypeStruct((_HOPS, _N, _D), jnp.float32),
    )(x, W)


def _combine_body(p_ref, o_ref):
    o_ref[...] = jnp.maximum(p_ref[0] + p_ref[1], 0.0)


def _combine(p):
    return pl.pallas_call(
        _combine_body,
        grid=(_N // _MB,),
        in_specs=[pl.BlockSpec((_NC, _MB, _D), lambda i: (0, i, 0))],
        out_specs=pl.BlockSpec((_MB, _D), lambda i: (i, 0)),
        out_shape=jax.ShapeDtypeStruct((_N, _D), jnp.float32),
    )(p)


@functools.partial(
    pl.kernel,
    out_type=jax.ShapeDtypeStruct((_NC, _N, _D), jnp.float32),
    mesh=plsc.VectorSubcoreMesh(core_axis_name="c", subcore_axis_name="s"),
    scratch_types=[
        pltpu.VMEM((2, _SB, 2, _C), jnp.int32),
        pltpu.VMEM((2, _SB, _C), jnp.float32),
        pltpu.VMEM((2, _C), jnp.int32),
        pltpu.VMEM((2, _C, _D), jnp.float32),
        pltpu.VMEM_SHARED((_N, _D), jnp.float32),
        pltpu.SemaphoreType.DMA,
        pltpu.SemaphoreType.DMA,
        pltpu.SemaphoreType.DMA,
        pltpu.SemaphoreType.DMA,
        pltpu.SemaphoreType.DMA,
        pltpu.SemaphoreType.DMA,
    ],
)
def _propagate(feat_hbm, crv_hbm, vals_hbm, out_hbm,
               crv, vals, ridx, rows, accum,
               semi0, semi1, semg0, semg1, sema0, sema1):
    semi = (semi0, semi1)
    semg = (semg0, semg1)
    sema = (sema0, sema1)

    c = lax.axis_index("c")
    s = lax.axis_index("s")
    w = s * _NC + c
    base = w * _NSUP

    def _zrow(e, carry):
        for k in range(_D // 16):
            rows[0, e, pl.ds(k * 16, 16)] = jnp.zeros((16,), jnp.float32)
        return carry

    lax.fori_loop(0, _C, _zrow, 0)

    nspan = _N // _C
    tail = _N - nspan * _C
    for j in range((nspan + _NS - 1) // _NS):
        idx = s + _NS * j

        @pl.when(idx < nspan)
        def _zero_span():
            off = pl.multiple_of(idx * _C, 8)
            pltpu.sync_copy(rows.at[0], accum.at[pl.ds(off, _C)])

    @pl.when(s == _NS - 1)
    def _zero_tail():
        pltpu.sync_copy(rows.at[0, pl.ds(0, tail)],
                        accum.at[pl.ds(nspan * _C, tail)])

    plsc.subcore_barrier()

    def _start_idx_load(sp, p):
        g = pl.multiple_of((base + sp) * _SB, 8)
        pltpu.async_copy(crv_hbm.at[pl.ds(g, _SB)], crv.at[p], semi[p])
        pltpu.async_copy(vals_hbm.at[pl.ds(g, _SB)], vals.at[p], semi[p])

    def _wait_idx_load(p):
        pltpu.make_async_copy(crv_hbm.at[pl.ds(0, _SB)], crv.at[p],
                              semi[p]).wait()
        pltpu.make_async_copy(vals_hbm.at[pl.ds(0, _SB)], vals.at[p],
                              semi[p]).wait()

    def _start_gather(p, k, b):
        pltpu.async_copy(feat_hbm.at[crv.at[p, k, 0]], rows.at[b],
                         semg[b])

    def _wait_gather(b):
        pltpu.make_async_copy(feat_hbm.at[crv.at[0, 0, 0]], rows.at[b],
                              semg[b]).wait()

    def _start_scatter(b):
        pltpu.async_copy(rows.at[b], accum.at[ridx.at[b]], sema[b],
                         add=True)

    def _wait_scatter(b):
        pltpu.make_async_copy(rows.at[b], accum.at[ridx.at[b]],
                              sema[b]).wait()

    def _for_parity(v, fn):
        @pl.when(v == 0)
        def _():
            fn(0)

        @pl.when(v != 0)
        def _():
            fn(1)

    _start_idx_load(0, 0)
    _wait_idx_load(0)
    _start_gather(0, 0, 0)

    def _chunk(i, carry):
        k = jnp.bitwise_and(i, _SB - 1)
        sp = lax.shift_right_logical(i, 3)
        p = jnp.bitwise_and(sp, 1)
        b = jnp.bitwise_and(i, 1)

        @pl.when(jnp.logical_and(k == 0, sp + 1 < _NSUP))
        def _():
            _for_parity(p, lambda q: _start_idx_load(sp + 1, 1 - q))

        _for_parity(b, _wait_gather)

        def _copy_ridx(e16, inner):
            sl = pl.ds(e16 * 16, 16)
            ridx[b, sl] = crv[p, k, 1, sl]
            return inner

        lax.fori_loop(0, _C // 16, _copy_ridx, 0)

        def _grp(g, inner):
            v16 = vals[p, k, pl.ds(g * 16, 16)]
            for j in range(16):
                e = g * 16 + j
                v = v16[j]
                for q in range(_D // 16):
                    sl = pl.ds(q * 16, 16)
                    rows[b, e, sl] = rows[b, e, sl] * v
            return inner

        lax.fori_loop(0, _C // 16, _grp, 0)
        _for_parity(b, _start_scatter)

        @pl.when(jnp.logical_and(k == _SB - 1, sp + 1 < _NSUP))
        def _():
            _for_parity(p, lambda q: _wait_idx_load(1 - q))

        j = i + 1

        @pl.when(j < _CPW)
        def _():
            kj = jnp.bitwise_and(j, _SB - 1)
            pj = jnp.bitwise_and(lax.shift_right_logical(j, 3), 1)
            bj = jnp.bitwise_and(j, 1)

            @pl.when(j >= 2)
            def _():
                _for_parity(bj, _wait_scatter)

            def _issue(q):
                def _issue2(bb):
                    pltpu.async_copy(feat_hbm.at[crv.at[q, kj, 0]],
                                     rows.at[bb], semg[bb])

                _for_parity(bj, _issue2)

            _for_parity(pj, _issue)
        return carry

    lax.fori_loop(0, _CPW, _chunk, 0)
    _wait_scatter(0)
    _wait_scatter(1)

    plsc.subcore_barrier()

    for j in range((nspan + _NS - 1) // _NS):
        idx = s + _NS * j

        @pl.when(idx < nspan)
        def _write_span():
            off = pl.multiple_of(idx * _C, 8)
            pltpu.sync_copy(accum.at[pl.ds(off, _C)],
                            out_hbm.at[c, pl.ds(off, _C)])

    @pl.when(s == _NS - 1)
    def _write_tail():
        pltpu.sync_copy(accum.at[pl.ds(nspan * _C, tail)],
                        out_hbm.at[c, pl.ds(nspan * _C, tail)])


def kernel(x, edge_index, edge_vals, W):
    feat = _feats(x, W).reshape(_HOPS * _N, _D)
    hop_off = (jnp.arange(_HOPS, dtype=jnp.int32) * _N)[:, None]
    col = (edge_index[:, 1, :] + hop_off).reshape(-1)
    row = edge_index[:, 0, :].reshape(-1)
    val = edge_vals.reshape(-1)
    pad = _EPAD - _TOTAL_E
    pad_ar = jnp.arange(pad, dtype=jnp.int32)
    col = jnp.concatenate([col, pad_ar % (_HOPS * _N)])
    row = jnp.concatenate([row, pad_ar % _N])
    val = jnp.concatenate([val, jnp.zeros((pad,), jnp.float32)])
    crv = jnp.stack([col.reshape(_NCHUNK, _C),
                     row.reshape(_NCHUNK, _C)], axis=1)
    partial = _propagate(feat, crv, val.reshape(_NCHUNK, _C))
    return _combine(partial)
